# baseline (speedup 1.0000x reference)
import functools
import math

import jax
import jax.numpy as jnp
from jax import lax
from jax.experimental import pallas as pl
from jax.experimental.pallas import tpu as pltpu
from jax.experimental.pallas import tpu_sc as plsc

_HIGHEST = lax.Precision.HIGHEST

_SC_CORES = 2
_SC_SUBCORES = 16
_SC_WORKERS = _SC_CORES * _SC_SUBCORES
_GATHER_CHUNK = 128
_PAD = 512


def _sc_gather_rows(table, idx_flat):
  h = table.shape[1]
  e = idx_flat.shape[0]
  assert e % _GATHER_CHUNK == 0
  n_chunks = e // _GATHER_CHUNK
  n_iters = (n_chunks + _SC_WORKERS - 1) // _SC_WORKERS

  mesh = plsc.VectorSubcoreMesh(core_axis_name="c", subcore_axis_name="s")

  @functools.partial(
      pl.kernel,
      mesh=mesh,
      out_type=jax.ShapeDtypeStruct((e, h), jnp.float32),
      scratch_types=[
          pltpu.VMEM((_GATHER_CHUNK,), jnp.int32),
          pltpu.VMEM((_GATHER_CHUNK, h), jnp.float32),
          pltpu.SemaphoreType.DMA,
      ],
  )
  def gather_kernel(table_hbm, idx_hbm, out_hbm, idx_v, rows_v, sem):
    wid = lax.axis_index("s") * _SC_CORES + lax.axis_index("c")

    def body(i, carry):
      cid = wid + i * _SC_WORKERS

      @pl.when(cid < n_chunks)
      def _():
        base = cid * _GATHER_CHUNK
        pltpu.sync_copy(idx_hbm.at[pl.ds(base, _GATHER_CHUNK)], idx_v)
        pltpu.async_copy(table_hbm.at[idx_v], rows_v, sem).wait()
        pltpu.sync_copy(rows_v, out_hbm.at[pl.ds(base, _GATHER_CHUNK)])

      return carry

    lax.fori_loop(0, n_iters, body, 0)

  return gather_kernel(table, idx_flat)


def _project(x, w):
  n, k = x.shape
  k2, m = w.shape
  b = 1024 if n % 1024 == 0 else 512

  def body(x_ref, w_ref, o_ref):
    o_ref[...] = jnp.dot(x_ref[...], w_ref[...], precision=_HIGHEST,
                         preferred_element_type=jnp.float32)

  return pl.pallas_call(
      body,
      grid=(n // b,),
      in_specs=[
          pl.BlockSpec((b, k), lambda i: (i, 0)),
          pl.BlockSpec((k2, m), lambda i: (0, 0)),
      ],
      out_specs=pl.BlockSpec((b, m), lambda i: (i, 0)),
      out_shape=jax.ShapeDtypeStruct((n, m), jnp.float32),
      compiler_params=pltpu.CompilerParams(
          dimension_semantics=("parallel",)),
  )(x, w)


def _sort_indices(time_t, idx_t):
  d, n = time_t.shape
  b = 512

  def body(tt_ref, idxt_ref, out_ref):
    tt = tt_ref[...]
    idxt = idxt_ref[...]
    te = tt[:, None, :]
    tj = tt[None, :, :]
    ii = lax.broadcasted_iota(jnp.int32, (d, d, b), 0)
    jj = lax.broadcasted_iota(jnp.int32, (d, d, b), 1)
    less = jnp.where((tj < te) | ((tj == te) & (jj < ii)),
                     jnp.int32(1), jnp.int32(0))
    rank_t = jnp.sum(less, axis=1)
    slot_t = jnp.int32(d - 1) - rank_t
    rows = []
    for r in range(d):
      picked = jnp.where(slot_t == r, idxt, jnp.int32(0))
      rows.append(jnp.sum(picked, axis=0, keepdims=True))
    tmax = jnp.max(tt, axis=0, keepdims=True)
    ties = jnp.sum(jnp.where(tt == tmax, jnp.int32(1), jnp.int32(0)),
                   axis=0, keepdims=True)
    picked = jnp.where(slot_t == ties - jnp.int32(1), idxt, jnp.int32(0))
    rows.append(jnp.sum(picked, axis=0, keepdims=True))
    out_ref[...] = jnp.concatenate(rows, axis=0)

  return pl.pallas_call(
      body,
      grid=(n // b,),
      in_specs=[
          pl.BlockSpec((d, b), lambda i: (0, i)),
          pl.BlockSpec((d, b), lambda i: (0, i)),
      ],
      out_specs=pl.BlockSpec((d + 1, b), lambda i: (0, i)),
      out_shape=jax.ShapeDtypeStruct((d + 1, n), jnp.int32),
      compiler_params=pltpu.CompilerParams(
          dimension_semantics=("parallel",)),
  )(time_t, idx_t)


def _reduce_update(mb_t, dst_h, emb_d, embk_d, w_agg, dst_raw, w_upd):
  d1, n, h = mb_t.shape
  d = d1 - 1
  b = 512
  inv_sqrt_h = 1.0 / math.sqrt(h)

  def body(mb_ref, dsth_ref, embd_ref, embkd_ref, wagg_ref,
           draw_ref, wupd_ref, out_ref):
    dsth = dsth_ref[...]
    last_emb = mb_ref[d]

    p_bn = lax.dot_general(dsth, embd_ref[...], (((1,), (1,)), ((), ())),
                           precision=_HIGHEST,
                           preferred_element_type=jnp.float32)

    dm_cols = []
    e1_cols = []
    for r in range(d):
      mbp = mb_ref[r]
      dm_cols.append(jnp.sum(mbp * dsth, axis=1, keepdims=True))
      e1_cols.append(jnp.sum(mbp * last_emb, axis=1, keepdims=True))
    dot_mb = jnp.concatenate(dm_cols, axis=1)

    e_ui = (p_bn + dot_mb) * inv_sqrt_h
    ex0 = jnp.exp(e_ui - jnp.max(e_ui, axis=1, keepdims=True))
    alpha = ex0 / jnp.sum(ex0, axis=1, keepdims=True)

    e1 = jnp.concatenate(e1_cols, axis=1) * inv_sqrt_h
    ex1 = jnp.exp(e1 - jnp.max(e1, axis=1, keepdims=True))
    alpha1 = ex1 / jnp.sum(ex1, axis=1, keepdims=True)

    h_long = lax.dot_general(alpha, embkd_ref[...], (((1,), (0,)), ((), ())),
                             precision=_HIGHEST,
                             preferred_element_type=jnp.float32)
    h_short = jnp.zeros((b, h), jnp.float32)
    for r in range(d):
      mbp = mb_ref[r]
      h_long = h_long + alpha[:, r:r + 1] * mbp
      h_short = h_short + alpha1[:, r:r + 1] * mbp

    agg = (jnp.dot(h_long, wagg_ref[0:h, :], precision=_HIGHEST,
                   preferred_element_type=jnp.float32) +
           jnp.dot(h_short, wagg_ref[h:2 * h, :], precision=_HIGHEST,
                   preferred_element_type=jnp.float32))
    out_ref[...] = jnp.tanh(
        jnp.dot(agg, wupd_ref[0:h, :], precision=_HIGHEST,
                preferred_element_type=jnp.float32) +
        jnp.dot(draw_ref[...], wupd_ref[h:2 * h, :], precision=_HIGHEST,
                preferred_element_type=jnp.float32))

  return pl.pallas_call(
      body,
      grid=(n // b,),
      in_specs=[
          pl.BlockSpec((d1, b, h), lambda i: (0, i, 0)),
          pl.BlockSpec((b, h), lambda i: (i, 0)),
          pl.BlockSpec((d, h), lambda i: (0, 0)),
          pl.BlockSpec((d, h), lambda i: (0, 0)),
          pl.BlockSpec((2 * h, h), lambda i: (0, 0)),
          pl.BlockSpec((b, h), lambda i: (i, 0)),
          pl.BlockSpec((2 * h, h), lambda i: (0, 0)),
      ],
      out_specs=pl.BlockSpec((b, h), lambda i: (i, 0)),
      out_shape=jax.ShapeDtypeStruct((n, h), jnp.float32),
      compiler_params=pltpu.CompilerParams(
          dimension_semantics=("parallel",)),
  )(mb_t, dst_h, emb_d, embk_d, w_agg, dst_raw, w_upd)


def kernel(user, item, by_src, by_time, pby_src, pby_time, W_user, W_recipe,
           W_user_update, W_recipe_update, W_agg_user, W_agg_recipe,
           user_date_emb, user_date_emb_k, recipe_date_emb,
           recipe_date_emb_k):
  nu, h = user.shape
  ni = item.shape[0]
  d = by_src.shape[1]
  npad_u = (-nu) % _PAD
  npad_i = (-ni) % _PAD

  user_p = jnp.pad(user, ((0, npad_u), (0, 0)))
  item_p = jnp.pad(item, ((0, npad_i), (0, 0)))
  by_time_p = jnp.pad(by_time, ((0, npad_u), (0, 0)))
  pby_time_p = jnp.pad(pby_time, ((0, npad_i), (0, 0)))
  by_src_t = jnp.pad(by_src, ((0, npad_u), (0, 0))).T.astype(jnp.int32)
  pby_src_t = jnp.pad(pby_src, ((0, npad_i), (0, 0))).T.astype(jnp.int32)

  user_h = _project(user_p, W_user)
  item_h = _project(item_p, W_recipe)

  idx_u_t = _sort_indices(by_time_p.T, by_src_t)
  idx_i_t = _sort_indices(pby_time_p.T, pby_src_t)
  nup = nu + npad_u
  nip = ni + npad_i
  mb_u = _sc_gather_rows(item_h, idx_u_t.reshape(-1)).reshape(d + 1, nup, h)
  mb_i = _sc_gather_rows(user_h, idx_i_t.reshape(-1)).reshape(d + 1, nip, h)

  user_out = _reduce_update(mb_u, user_h, user_date_emb[:d],
                            user_date_emb_k[:d], W_agg_user, user_p,
                            W_user_update)
  item_out = _reduce_update(mb_i, item_h, recipe_date_emb[:d],
                            recipe_date_emb_k[:d], W_agg_recipe, item_p,
                            W_recipe_update)
  return (user_out[:nu], item_out[:ni])


# speedup vs baseline: 1.1029x; 1.1029x over previous
import functools
import math

import jax
import jax.numpy as jnp
from jax import lax
from jax.experimental import pallas as pl
from jax.experimental.pallas import tpu as pltpu
from jax.experimental.pallas import tpu_sc as plsc

_HIGHEST = lax.Precision.HIGHEST

_SC_CORES = 2
_SC_SUBCORES = 16
_SC_WORKERS = _SC_CORES * _SC_SUBCORES
_GATHER_CHUNK = 128
_PAD = 512


_GRP = 3


def _sc_gather_two(table_a, idx_a, table_b, idx_b):
  h = table_a.shape[1]
  e = idx_a.shape[0]
  assert e == idx_b.shape[0]
  n_chunks = e // _GATHER_CHUNK
  per_w = n_chunks // (_SC_WORKERS // 2)
  assert per_w * (_SC_WORKERS // 2) == n_chunks and per_w % _GRP == 0
  n_groups = per_w // _GRP
  rows_grp = _GRP * _GATHER_CHUNK
  idx_per_w = per_w * _GATHER_CHUNK

  mesh = plsc.VectorSubcoreMesh(core_axis_name="c", subcore_axis_name="s")

  @functools.partial(
      pl.kernel,
      mesh=mesh,
      out_type=(jax.ShapeDtypeStruct((e, h), jnp.float32),
                jax.ShapeDtypeStruct((e, h), jnp.float32)),
      scratch_types=[
          pltpu.VMEM((idx_per_w,), jnp.int32),
          pltpu.VMEM((2, rows_grp, h), jnp.float32),
          pltpu.SemaphoreType.DMA,
          pltpu.SemaphoreType.DMA,
          pltpu.SemaphoreType.DMA,
          pltpu.SemaphoreType.DMA,
      ],
  )
  def gather_kernel(ta_hbm, ia_hbm, tb_hbm, ib_hbm, oa_hbm, ob_hbm,
                    idx_v, rows_v, sg0, sg1, ss0, ss1):
    wid = lax.axis_index("s") * _SC_CORES + lax.axis_index("c")

    def run_side(table_hbm, idx_hbm, out_hbm, lw):
      base_el = lw * idx_per_w
      pltpu.sync_copy(idx_hbm.at[pl.ds(base_el, idx_per_w)], idx_v)

      def start_gathers(g, buf, sem):
        for s in range(_GRP):
          off = g * rows_grp + s * _GATHER_CHUNK
          pltpu.async_copy(
              table_hbm.at[idx_v.at[pl.ds(off, _GATHER_CHUNK)]],
              rows_v.at[buf, pl.ds(s * _GATHER_CHUNK, _GATHER_CHUNK)],
              sem)

      def drain_gathers(buf, sem):
        for s in range(_GRP):
          pltpu.make_async_copy(
              table_hbm.at[pl.ds(0, _GATHER_CHUNK)],
              rows_v.at[buf, pl.ds(s * _GATHER_CHUNK, _GATHER_CHUNK)],
              sem).wait()

      def store(g, buf, sem):
        pltpu.async_copy(rows_v.at[buf],
                         out_hbm.at[pl.ds(base_el + g * rows_grp, rows_grp)],
                         sem)

      def wait_store(buf, sem):
        pltpu.make_async_copy(table_hbm.at[pl.ds(0, rows_grp)],
                              rows_v.at[buf], sem).wait()

      start_gathers(0, 0, sg0)
      start_gathers(1, 1, sg1)

      def body(j, carry):
        g0 = j * 2
        g1 = j * 2 + 1
        drain_gathers(0, sg0)
        store(g0, 0, ss0)

        @pl.when(g0 + 2 < n_groups)
        def _():
          wait_store(0, ss0)
          start_gathers(g0 + 2, 0, sg0)

        drain_gathers(1, sg1)
        store(g1, 1, ss1)

        @pl.when(g1 + 2 < n_groups)
        def _():
          wait_store(1, ss1)
          start_gathers(g1 + 2, 1, sg1)

        return carry

      lax.fori_loop(0, n_groups // 2, body, 0)
      if n_groups % 2:
        drain_gathers(0, sg0)
        store(n_groups - 1, 0, ss0)
      wait_store(0, ss0)
      wait_store(1, ss1)

    @pl.when(wid < _SC_WORKERS // 2)
    def _():
      run_side(ta_hbm, ia_hbm, oa_hbm, wid)

    @pl.when(wid >= _SC_WORKERS // 2)
    def _():
      run_side(tb_hbm, ib_hbm, ob_hbm, wid - _SC_WORKERS // 2)

  return gather_kernel(table_a, idx_a, table_b, idx_b)


def _project(x, w):
  n, k = x.shape
  k2, m = w.shape
  b = 1024 if n % 1024 == 0 else 512

  def body(x_ref, w_ref, o_ref):
    o_ref[...] = jnp.dot(x_ref[...], w_ref[...], precision=_HIGHEST,
                         preferred_element_type=jnp.float32)

  return pl.pallas_call(
      body,
      grid=(n // b,),
      in_specs=[
          pl.BlockSpec((b, k), lambda i: (i, 0)),
          pl.BlockSpec((k2, m), lambda i: (0, 0)),
      ],
      out_specs=pl.BlockSpec((b, m), lambda i: (i, 0)),
      out_shape=jax.ShapeDtypeStruct((n, m), jnp.float32),
      compiler_params=pltpu.CompilerParams(
          dimension_semantics=("parallel",)),
  )(x, w)


def _sort_indices(time_t, idx_t):
  d, n = time_t.shape
  b = 512

  def body(tt_ref, idxt_ref, out_ref):
    tt = tt_ref[...]
    idxt = idxt_ref[...]
    te = tt[:, None, :]
    tj = tt[None, :, :]
    ii = lax.broadcasted_iota(jnp.int32, (d, d, b), 0)
    jj = lax.broadcasted_iota(jnp.int32, (d, d, b), 1)
    less = jnp.where((tj < te) | ((tj == te) & (jj < ii)),
                     jnp.int32(1), jnp.int32(0))
    rank_t = jnp.sum(less, axis=1)
    slot_t = jnp.int32(d - 1) - rank_t
    rows = []
    for r in range(d):
      picked = jnp.where(slot_t == r, idxt, jnp.int32(0))
      rows.append(jnp.sum(picked, axis=0, keepdims=True))
    tmax = jnp.max(tt, axis=0, keepdims=True)
    ties = jnp.sum(jnp.where(tt == tmax, jnp.int32(1), jnp.int32(0)),
                   axis=0, keepdims=True)
    picked = jnp.where(slot_t == ties - jnp.int32(1), idxt, jnp.int32(0))
    rows.append(jnp.sum(picked, axis=0, keepdims=True))
    out_ref[...] = jnp.concatenate(rows, axis=0)

  return pl.pallas_call(
      body,
      grid=(n // b,),
      in_specs=[
          pl.BlockSpec((d, b), lambda i: (0, i)),
          pl.BlockSpec((d, b), lambda i: (0, i)),
      ],
      out_specs=pl.BlockSpec((d + 1, b), lambda i: (0, i)),
      out_shape=jax.ShapeDtypeStruct((d + 1, n), jnp.int32),
      compiler_params=pltpu.CompilerParams(
          dimension_semantics=("parallel",)),
  )(time_t, idx_t)


def _reduce_update(mb_t, dst_h, emb_d, embk_d, w_agg, dst_raw, w_upd):
  d1, n, h = mb_t.shape
  d = d1 - 1
  b = 512
  inv_sqrt_h = 1.0 / math.sqrt(h)

  def body(mb_ref, dsth_ref, embd_ref, embkd_ref, wagg_ref,
           draw_ref, wupd_ref, out_ref):
    dsth = dsth_ref[...]
    last_emb = mb_ref[d]

    p_bn = lax.dot_general(dsth, embd_ref[...], (((1,), (1,)), ((), ())),
                           precision=_HIGHEST,
                           preferred_element_type=jnp.float32)

    dm_cols = []
    e1_cols = []
    for r in range(d):
      mbp = mb_ref[r]
      dm_cols.append(jnp.sum(mbp * dsth, axis=1, keepdims=True))
      e1_cols.append(jnp.sum(mbp * last_emb, axis=1, keepdims=True))
    dot_mb = jnp.concatenate(dm_cols, axis=1)

    e_ui = (p_bn + dot_mb) * inv_sqrt_h
    ex0 = jnp.exp(e_ui - jnp.max(e_ui, axis=1, keepdims=True))
    alpha = ex0 / jnp.sum(ex0, axis=1, keepdims=True)

    e1 = jnp.concatenate(e1_cols, axis=1) * inv_sqrt_h
    ex1 = jnp.exp(e1 - jnp.max(e1, axis=1, keepdims=True))
    alpha1 = ex1 / jnp.sum(ex1, axis=1, keepdims=True)

    h_long = lax.dot_general(alpha, embkd_ref[...], (((1,), (0,)), ((), ())),
                             precision=_HIGHEST,
                             preferred_element_type=jnp.float32)
    h_short = jnp.zeros((b, h), jnp.float32)
    for r in range(d):
      mbp = mb_ref[r]
      h_long = h_long + alpha[:, r:r + 1] * mbp
      h_short = h_short + alpha1[:, r:r + 1] * mbp

    agg = (jnp.dot(h_long, wagg_ref[0:h, :], precision=_HIGHEST,
                   preferred_element_type=jnp.float32) +
           jnp.dot(h_short, wagg_ref[h:2 * h, :], precision=_HIGHEST,
                   preferred_element_type=jnp.float32))
    out_ref[...] = jnp.tanh(
        jnp.dot(agg, wupd_ref[0:h, :], precision=_HIGHEST,
                preferred_element_type=jnp.float32) +
        jnp.dot(draw_ref[...], wupd_ref[h:2 * h, :], precision=_HIGHEST,
                preferred_element_type=jnp.float32))

  return pl.pallas_call(
      body,
      grid=(n // b,),
      in_specs=[
          pl.BlockSpec((d1, b, h), lambda i: (0, i, 0)),
          pl.BlockSpec((b, h), lambda i: (i, 0)),
          pl.BlockSpec((d, h), lambda i: (0, 0)),
          pl.BlockSpec((d, h), lambda i: (0, 0)),
          pl.BlockSpec((2 * h, h), lambda i: (0, 0)),
          pl.BlockSpec((b, h), lambda i: (i, 0)),
          pl.BlockSpec((2 * h, h), lambda i: (0, 0)),
      ],
      out_specs=pl.BlockSpec((b, h), lambda i: (i, 0)),
      out_shape=jax.ShapeDtypeStruct((n, h), jnp.float32),
      compiler_params=pltpu.CompilerParams(
          dimension_semantics=("parallel",)),
  )(mb_t, dst_h, emb_d, embk_d, w_agg, dst_raw, w_upd)


def kernel(user, item, by_src, by_time, pby_src, pby_time, W_user, W_recipe,
           W_user_update, W_recipe_update, W_agg_user, W_agg_recipe,
           user_date_emb, user_date_emb_k, recipe_date_emb,
           recipe_date_emb_k):
  nu, h = user.shape
  ni = item.shape[0]
  d = by_src.shape[1]
  npad_u = (-nu) % _PAD
  npad_i = (-ni) % _PAD

  user_p = jnp.pad(user, ((0, npad_u), (0, 0)))
  item_p = jnp.pad(item, ((0, npad_i), (0, 0)))
  by_time_p = jnp.pad(by_time, ((0, npad_u), (0, 0)))
  pby_time_p = jnp.pad(pby_time, ((0, npad_i), (0, 0)))
  by_src_t = jnp.pad(by_src, ((0, npad_u), (0, 0))).T.astype(jnp.int32)
  pby_src_t = jnp.pad(pby_src, ((0, npad_i), (0, 0))).T.astype(jnp.int32)

  user_h = _project(user_p, W_user)
  item_h = _project(item_p, W_recipe)

  idx_u_t = _sort_indices(by_time_p.T, by_src_t)
  idx_i_t = _sort_indices(pby_time_p.T, pby_src_t)
  nup = nu + npad_u
  nip = ni + npad_i
  mb_u_flat, mb_i_flat = _sc_gather_two(
      item_h, idx_u_t.reshape(-1), user_h, idx_i_t.reshape(-1))
  mb_u = mb_u_flat.reshape(d + 1, nup, h)
  mb_i = mb_i_flat.reshape(d + 1, nip, h)

  user_out = _reduce_update(mb_u, user_h, user_date_emb[:d],
                            user_date_emb_k[:d], W_agg_user, user_p,
                            W_user_update)
  item_out = _reduce_update(mb_i, item_h, recipe_date_emb[:d],
                            recipe_date_emb_k[:d], W_agg_recipe, item_p,
                            W_recipe_update)
  return (user_out[:nu], item_out[:ni])
